# baseline (speedup 1.0000x reference)
import numpy as np
import jax
import jax.numpy as jnp
from jax import lax
from jax.experimental import pallas as pl
from jax.experimental.pallas import tpu as pltpu
from jax.experimental.pallas import tpu_sc as plsc

_PATCH = 32
_NPH = 16
_NUM_PATCHES = _NPH * _NPH
_NUM_MASKED = 154
_C, _H, _W = 192, 512, 512
_NW = 32
_CPW = _C // _NW

_MASKED_IDS = np.array([
    121, 35, 130, 148, 197, 45, 176, 179, 139, 188, 99, 144, 152, 189, 31,
    112, 85, 63, 117, 174, 114, 254, 82, 65, 7, 4, 101, 102, 78, 163, 157,
    183, 29, 240, 177, 108, 83, 129, 212, 44, 211, 16, 58, 123, 37, 111, 19,
    61, 2, 142, 34, 156, 5, 90, 175, 167, 251, 110, 72, 155, 178, 219, 153,
    30, 42, 186, 246, 3, 70, 67, 223, 39, 56, 192, 169, 218, 195, 173, 245,
    241, 69, 80, 22, 6, 199, 118, 235, 54, 77, 147, 18, 249, 10, 11, 234, 53,
    236, 94, 32, 217, 159, 15, 184, 49, 137, 50, 138, 20, 237, 253, 185, 43,
    92, 8, 140, 233, 24, 81, 239, 96, 154, 135, 160, 106, 128, 191, 9, 200,
    40, 187, 71, 248, 164, 207, 93, 59, 201, 158, 210, 75, 131, 97, 66, 25,
    196, 242, 206, 243, 238, 73, 13, 52, 203, 202], dtype=np.int32)
_MASKED2D = np.zeros((_NUM_PATCHES,), dtype=bool)
_MASKED2D[_MASKED_IDS] = True
_MASKED2D = _MASKED2D.reshape(_NPH, _NPH)

_idx_pad = np.full((_NUM_PATCHES,), -1, dtype=np.int32)
_idx_pad[:_NUM_MASKED] = _MASKED_IDS
_IDX_ROW = _idx_pad.reshape(1, _NUM_PATCHES)
_IDX_COL = _idx_pad.reshape(_NUM_PATCHES, 1)


def _runs(row, value):
    out, j = [], 0
    while j < _NPH:
        if row[j] == value:
            k = j
            while k < _NPH and row[k] == value:
                k += 1
            out.append((j * _PATCH, (k - j) * _PATCH))
            j = k
        else:
            j += 1
    return out


_KEEP_RUNS = [_runs(_MASKED2D[ph], False) for ph in range(_NPH)]
_ZERO_CHUNKS = [
    [w0 + 16 * k for (w0, wl) in _runs(_MASKED2D[ph], True) for k in range(wl // 16)]
    for ph in range(_NPH)
]


def _mask_kernel(idx_row_ref, idx_col_ref, keep_ref, maskb_ref):
    n = _NUM_PATCHES
    hh = lax.broadcasted_iota(jnp.int32, (_H, n), 0) // _PATCH
    ph = idx_row_ref[...] // _NPH
    r = (hh == ph).astype(jnp.float32)
    ww = lax.broadcasted_iota(jnp.int32, (n, _W), 1) // _PATCH
    pw = idx_col_ref[...] % _NPH
    c = (ww == pw).astype(jnp.float32)
    mask = jnp.dot(r, c, preferred_element_type=jnp.float32)
    keep_ref[...] = 1.0 - mask
    maskb_ref[...] = (mask > 0.5)[None, :, :]


def _sc_body(img_hbm, out_hbm, buf0, buf1, gsem, wsem0, wsem1):
    wid = lax.axis_index("s") * 2 + lax.axis_index("c")
    c0 = wid * _CPW
    bufs = (buf0, buf1)
    wsems = (wsem0, wsem1)
    pending = [None, None]
    zeros16 = jnp.zeros((16,), jnp.float32)
    for ph in range(_NPH):
        h0 = ph * _PATCH
        chunks = _ZERO_CHUNKS[ph]
        for i in range(_CPW):
            b = i % 2
            buf = bufs[b]
            c = c0 + i
            if pending[b] is not None:
                pending[b].wait()
                pending[b] = None
            if i < 2 and chunks:
                def _zrow(r, _, buf=buf, chunks=chunks):
                    for w0 in chunks:
                        buf[r, pl.ds(w0, 16)] = zeros16
                    return ()
                lax.fori_loop(0, _PATCH, _zrow, ())
            handles = [
                pltpu.async_copy(
                    img_hbm.at[c, pl.ds(h0, _PATCH), pl.ds(w0, wl)],
                    buf.at[:, pl.ds(w0, wl)],
                    gsem,
                )
                for (w0, wl) in _KEEP_RUNS[ph]
            ]
            for hnd in handles:
                hnd.wait()
            pending[b] = pltpu.async_copy(
                buf, out_hbm.at[c, pl.ds(h0, _PATCH), :], wsems[b]
            )
    for b in (0, 1):
        if pending[b] is not None:
            pending[b].wait()


_sc_masked = pl.kernel(
    _sc_body,
    out_type=jax.ShapeDtypeStruct((_C, _H, _W), jnp.float32),
    mesh=plsc.VectorSubcoreMesh(core_axis_name="c", subcore_axis_name="s"),
    scratch_types=[
        pltpu.VMEM((_PATCH, _W), jnp.float32),
        pltpu.VMEM((_PATCH, _W), jnp.float32),
        pltpu.SemaphoreType.DMA,
        pltpu.SemaphoreType.DMA,
        pltpu.SemaphoreType.DMA,
    ],
    compiler_params=pltpu.CompilerParams(use_tc_tiling_on_sc=False),
)


def kernel(image):
    _, mask_full = pl.pallas_call(
        _mask_kernel,
        out_shape=(
            jax.ShapeDtypeStruct((_H, _W), jnp.float32),
            jax.ShapeDtypeStruct((1, _H, _W), jnp.bool_),
        ),
    )(jnp.asarray(_IDX_ROW), jnp.asarray(_IDX_COL))
    masked = _sc_masked(image)
    return masked, mask_full


# speedup vs baseline: 1.7520x; 1.7520x over previous
import numpy as np
import jax
import jax.numpy as jnp
from jax import lax
from jax.experimental import pallas as pl
from jax.experimental.pallas import tpu as pltpu

_PATCH = 32
_NPH = 16
_NUM_PATCHES = _NPH * _NPH
_NUM_MASKED = 154
_C, _H, _W = 192, 512, 512
_C_BLK = 8
_WBLK = 128
_NWB = _W // _WBLK

_MASKED_IDS = np.array([
    121, 35, 130, 148, 197, 45, 176, 179, 139, 188, 99, 144, 152, 189, 31,
    112, 85, 63, 117, 174, 114, 254, 82, 65, 7, 4, 101, 102, 78, 163, 157,
    183, 29, 240, 177, 108, 83, 129, 212, 44, 211, 16, 58, 123, 37, 111, 19,
    61, 2, 142, 34, 156, 5, 90, 175, 167, 251, 110, 72, 155, 178, 219, 153,
    30, 42, 186, 246, 3, 70, 67, 223, 39, 56, 192, 169, 218, 195, 173, 245,
    241, 69, 80, 22, 6, 199, 118, 235, 54, 77, 147, 18, 249, 10, 11, 234, 53,
    236, 94, 32, 217, 159, 15, 184, 49, 137, 50, 138, 20, 237, 253, 185, 43,
    92, 8, 140, 233, 24, 81, 239, 96, 154, 135, 160, 106, 128, 191, 9, 200,
    40, 187, 71, 248, 164, 207, 93, 59, 201, 158, 210, 75, 131, 97, 66, 25,
    196, 242, 206, 243, 238, 73, 13, 52, 203, 202], dtype=np.int32)
_MASKED2D = np.zeros((_NUM_PATCHES,), dtype=bool)
_MASKED2D[_MASKED_IDS] = True
_MASKED2D = _MASKED2D.reshape(_NPH, _NPH)

_idx_pad = np.full((_NUM_PATCHES,), -1, dtype=np.int32)
_idx_pad[:_NUM_MASKED] = _MASKED_IDS
_IDX_ROW = _idx_pad.reshape(1, _NUM_PATCHES)
_IDX_COL = _idx_pad.reshape(_NUM_PATCHES, 1)

_FULLY_MASKED = _MASKED2D.reshape(_NPH, _NWB, _WBLK // _PATCH).all(axis=2)

_JMAP = []
for _w in range(_NWB):
    jm = [None] * _NPH
    live = [j for j in range(_NPH) if not _FULLY_MASKED[j, _w]]
    for _j in range(_NPH):
        if not _FULLY_MASKED[_j, _w]:
            jm[_j] = _j
        else:
            prev = [j for j in live if j < _j]
            nxt = [j for j in live if j > _j]
            jm[_j] = prev[-1] if prev else nxt[0]
    _JMAP.append(jm)


def _mask_kernel(idx_row_ref, idx_col_ref, keep_ref, maskb_ref):
    n = _NUM_PATCHES
    hh = lax.broadcasted_iota(jnp.int32, (_H, n), 0) // _PATCH
    ph = idx_row_ref[...] // _NPH
    r = (hh == ph).astype(jnp.float32)
    ww = lax.broadcasted_iota(jnp.int32, (n, _W), 1) // _PATCH
    pw = idx_col_ref[...] % _NPH
    c = (ww == pw).astype(jnp.float32)
    mask = jnp.dot(r, c, preferred_element_type=jnp.float32)
    keep_ref[...] = 1.0 - mask
    maskb_ref[...] = (mask > 0.5)[None, :, :]


def _mul_kernel(img0, img1, img2, img3, keep_ref, out_ref):
    j = pl.program_id(1)
    keep = keep_ref[pl.ds(j * _PATCH, _PATCH), :]
    imgs = (img0, img1, img2, img3)
    for w in range(_NWB):
        out_ref[:, :, w * _WBLK:(w + 1) * _WBLK] = (
            imgs[w][...] * keep[None, :, w * _WBLK:(w + 1) * _WBLK]
        )


def _img_spec(w):
    jm = _JMAP[w]

    def index_map(i, j):
        jj = sum((j == k) * int(jm[k]) for k in range(_NPH))
        return (i, jj, w)

    return pl.BlockSpec((_C_BLK, _PATCH, _WBLK), index_map)


def kernel(image):
    keep, mask_full = pl.pallas_call(
        _mask_kernel,
        out_shape=(
            jax.ShapeDtypeStruct((_H, _W), jnp.float32),
            jax.ShapeDtypeStruct((1, _H, _W), jnp.bool_),
        ),
    )(jnp.asarray(_IDX_ROW), jnp.asarray(_IDX_COL))

    masked = pl.pallas_call(
        _mul_kernel,
        grid=(_C // _C_BLK, _NPH),
        in_specs=[_img_spec(w) for w in range(_NWB)]
        + [pl.BlockSpec((_H, _W), lambda i, j: (0, 0))],
        out_specs=pl.BlockSpec((_C_BLK, _PATCH, _W), lambda i, j: (i, j, 0)),
        out_shape=jax.ShapeDtypeStruct((_C, _H, _W), jnp.float32),
        compiler_params=pltpu.CompilerParams(
            dimension_semantics=("parallel", "arbitrary"),
        ),
    )(image, image, image, image, keep)
    return masked, mask_full


# speedup vs baseline: 4.0817x; 2.3298x over previous
import numpy as np
import jax
import jax.numpy as jnp
from jax import lax
from jax.experimental import pallas as pl
from jax.experimental.pallas import tpu as pltpu

_PATCH = 32
_NPH = 16
_NUM_PATCHES = _NPH * _NPH
_NUM_MASKED = 154
_C_BLK = 8

_perm = np.asarray(jax.random.permutation(jax.random.key(42), _NUM_PATCHES))
_idx_pad = np.full((_NUM_PATCHES,), -1, dtype=np.int32)
_idx_pad[:_NUM_MASKED] = _perm[:_NUM_MASKED].astype(np.int32)
_IDX_ROW = _idx_pad.reshape(1, _NUM_PATCHES)
_IDX_COL = _idx_pad.reshape(_NUM_PATCHES, 1)


def _mask_kernel(idx_row_ref, idx_col_ref, keep_ref, maskb_ref):
    n = _NUM_PATCHES
    hh = lax.broadcasted_iota(jnp.int32, (512, n), 0) // _PATCH
    ph = idx_row_ref[...] // _NPH
    r = (hh == ph).astype(jnp.float32)
    ww = lax.broadcasted_iota(jnp.int32, (n, 512), 1) // _PATCH
    pw = idx_col_ref[...] % _NPH
    c = (ww == pw).astype(jnp.float32)
    mask = jnp.dot(r, c, preferred_element_type=jnp.float32)
    keep_ref[...] = 1.0 - mask
    maskb_ref[...] = (mask > 0.5)[None, :, :]


def _mul_kernel(img_ref, keep_ref, out_ref):
    out_ref[...] = img_ref[...] * keep_ref[...][None, :, :]


def kernel(image):
    C, H, W = image.shape
    keep, mask_full = pl.pallas_call(
        _mask_kernel,
        out_shape=(
            jax.ShapeDtypeStruct((H, W), jnp.float32),
            jax.ShapeDtypeStruct((1, H, W), jnp.bool_),
        ),
    )(jnp.asarray(_IDX_ROW), jnp.asarray(_IDX_COL))

    masked = pl.pallas_call(
        _mul_kernel,
        grid=(C // _C_BLK,),
        in_specs=[
            pl.BlockSpec((_C_BLK, H, W), lambda i: (i, 0, 0)),
            pl.BlockSpec((H, W), lambda i: (0, 0)),
        ],
        out_specs=pl.BlockSpec((_C_BLK, H, W), lambda i: (i, 0, 0)),
        out_shape=jax.ShapeDtypeStruct((C, H, W), jnp.float32),
        compiler_params=pltpu.CompilerParams(
            dimension_semantics=("parallel",),
        ),
    )(image, keep)
    return masked, mask_full


# speedup vs baseline: 4.1433x; 1.0151x over previous
import numpy as np
import jax
import jax.numpy as jnp
from jax import lax
from jax.experimental import pallas as pl
from jax.experimental.pallas import tpu as pltpu

_PATCH = 32
_NPH = 16
_NUM_PATCHES = _NPH * _NPH
_NUM_MASKED = 154
_C, _H, _W = 192, 512, 512
_C_BLK = 8

_MASKED_IDS = np.array([
    121, 35, 130, 148, 197, 45, 176, 179, 139, 188, 99, 144, 152, 189, 31,
    112, 85, 63, 117, 174, 114, 254, 82, 65, 7, 4, 101, 102, 78, 163, 157,
    183, 29, 240, 177, 108, 83, 129, 212, 44, 211, 16, 58, 123, 37, 111, 19,
    61, 2, 142, 34, 156, 5, 90, 175, 167, 251, 110, 72, 155, 178, 219, 153,
    30, 42, 186, 246, 3, 70, 67, 223, 39, 56, 192, 169, 218, 195, 173, 245,
    241, 69, 80, 22, 6, 199, 118, 235, 54, 77, 147, 18, 249, 10, 11, 234, 53,
    236, 94, 32, 217, 159, 15, 184, 49, 137, 50, 138, 20, 237, 253, 185, 43,
    92, 8, 140, 233, 24, 81, 239, 96, 154, 135, 160, 106, 128, 191, 9, 200,
    40, 187, 71, 248, 164, 207, 93, 59, 201, 158, 210, 75, 131, 97, 66, 25,
    196, 242, 206, 243, 238, 73, 13, 52, 203, 202], dtype=np.int32)
_idx_pad = np.full((_NUM_PATCHES,), -1, dtype=np.int32)
_idx_pad[:_NUM_MASKED] = _MASKED_IDS
_IDX_ROW = _idx_pad.reshape(1, _NUM_PATCHES)
_IDX_COL = _idx_pad.reshape(_NUM_PATCHES, 1)


def _fused_kernel(idx_row_ref, idx_col_ref, img_ref, out_ref, maskb_ref):
    n = _NUM_PATCHES
    u = (lax.broadcasted_iota(jnp.int32, (_NPH, n), 0)
         == idx_row_ref[...] // _NPH).astype(jnp.float32)
    v = (lax.broadcasted_iota(jnp.int32, (n, _NPH), 1)
         == idx_col_ref[...] % _NPH).astype(jnp.float32)
    pm16 = jnp.dot(u, v, preferred_element_type=jnp.float32)
    e = (lax.broadcasted_iota(jnp.int32, (_H, _NPH), 0) // _PATCH
         == lax.broadcasted_iota(jnp.int32, (_H, _NPH), 1)
         ).astype(jnp.float32)
    e2 = (lax.broadcasted_iota(jnp.int32, (_NPH, _W), 1) // _PATCH
          == lax.broadcasted_iota(jnp.int32, (_NPH, _W), 0)
          ).astype(jnp.float32)
    mask = jnp.dot(e, jnp.dot(pm16, e2, preferred_element_type=jnp.float32),
                   preferred_element_type=jnp.float32)
    out_ref[...] = img_ref[...] * (1.0 - mask)[None, :, :]
    maskb_ref[...] = (mask > 0.5)[None, :, :]


def kernel(image):
    masked, mask_full = pl.pallas_call(
        _fused_kernel,
        grid=(_C // _C_BLK,),
        in_specs=[
            pl.BlockSpec((1, _NUM_PATCHES), lambda i: (0, 0)),
            pl.BlockSpec((_NUM_PATCHES, 1), lambda i: (0, 0)),
            pl.BlockSpec((_C_BLK, _H, _W), lambda i: (i, 0, 0)),
        ],
        out_specs=(
            pl.BlockSpec((_C_BLK, _H, _W), lambda i: (i, 0, 0)),
            pl.BlockSpec((1, _H, _W), lambda i: (0, 0, 0)),
        ),
        out_shape=(
            jax.ShapeDtypeStruct((_C, _H, _W), jnp.float32),
            jax.ShapeDtypeStruct((1, _H, _W), jnp.bool_),
        ),
        compiler_params=pltpu.CompilerParams(
            dimension_semantics=("parallel",),
        ),
    )(jnp.asarray(_IDX_ROW), jnp.asarray(_IDX_COL), image)
    return masked, mask_full
